# baseline (speedup 1.0000x reference)
import functools

import jax
import jax.numpy as jnp
import numpy as np
from jax import lax
from jax.experimental import pallas as pl
from jax.experimental.pallas import tpu as pltpu
from jax.experimental.pallas import tpu_sc as plsc

S = 30.0
M = 0.5
ARC_START_EPOCH = 1
COS_M = float(np.cos(M))
SIN_M = float(np.sin(M))
BORDER = float(np.cos(np.pi - M))

_LOG1P_COEF = [float(c) for c in
               np.polynomial.chebyshev.Chebyshev.interpolate(
                   np.log1p, 7, domain=[0.0, 1.0])
               .convert(kind=np.polynomial.Polynomial).coef]

NW = 32
L = 16


def _sc_loss_tile(c, t, arc_b, scale_v):
    x = jnp.maximum(1.0 - c * c, 0.0)
    i = lax.bitcast_convert_type(x, jnp.int32)
    i = jnp.int32(0x5F3759DF) - lax.shift_right_arithmetic(i, 1)
    y = lax.bitcast_convert_type(i, jnp.float32)
    hx = 0.5 * x
    y = y * (1.5 - hx * y * y)
    y = y * (1.5 - hx * y * y)
    s = x * y

    phai = c * COS_M - s * SIN_M
    phai = jnp.where(c > BORDER, phai, -2.0 - phai)

    tmask = t != 0.0
    inner = jnp.where(arc_b, phai, c)
    v = scale_v * jnp.where(tmask, -inner, c)

    q = jnp.exp(jnp.minimum(v, -v))
    p = _LOG1P_COEF[7]
    for k in range(6, -1, -1):
        p = p * q + _LOG1P_COEF[k]
    sp = jnp.maximum(v, 0.0) + p
    loss = jnp.exp(2.0 * (v - sp)) * sp
    corr = jnp.where(v < 0.0, 1.0, 0.0)
    return loss, corr


def _sc_body(arc_hbm, fc_hbm, lb_hbm, loss_out, corr_out,
             fcv0, lbv0, fcv1, lbv1, pvec, stage, sem0, sem1,
             *, per_w, ch):
    cid = lax.axis_index("c")
    sid = lax.axis_index("s")
    wid = sid * 2 + cid
    base = wid * per_w
    nch = per_w // ch
    nvec = ch // L

    pltpu.sync_copy(arc_hbm, pvec)
    arc_b = pvec[...] != 0.0
    scale_v = jnp.where(arc_b, jnp.float32(S), jnp.float32(1.0))

    bufs = ((fcv0, lbv0, sem0), (fcv1, lbv1, sem1))

    def start(g, b):
        off = base + g * ch
        pltpu.async_copy(fc_hbm.at[pl.ds(off, ch)], bufs[b][0], bufs[b][2])
        pltpu.async_copy(lb_hbm.at[pl.ds(off, ch)], bufs[b][1], bufs[b][2])

    def wait(b):
        pltpu.make_async_copy(fc_hbm.at[pl.ds(0, ch)], bufs[b][0],
                              bufs[b][2]).wait()
        pltpu.make_async_copy(lb_hbm.at[pl.ds(0, ch)], bufs[b][1],
                              bufs[b][2]).wait()

    def compute(b, lacc, cacc):
        fcv, lbv = bufs[b][0], bufs[b][1]

        def col_step(j, carry2):
            la2, ca2 = carry2
            c = fcv[pl.ds(j * L, L)]
            t = lbv[pl.ds(j * L, L)]
            lo, co = _sc_loss_tile(c, t, arc_b, scale_v)
            return la2 + lo, ca2 + co

        return lax.fori_loop(0, nvec, col_step, (lacc, cacc), unroll=4)

    zero = jnp.zeros((L,), jnp.float32)
    start(0, 0)

    def chunk_pair(i2, carry):
        lacc, cacc = carry
        g = i2 * 2
        wait(0)

        @pl.when(g + 1 < nch)
        def _():
            start(g + 1, 1)

        lacc, cacc = compute(0, lacc, cacc)
        wait(1)

        @pl.when(g + 2 < nch)
        def _():
            start(g + 2, 0)

        lacc, cacc = compute(1, lacc, cacc)
        return lacc, cacc

    lacc, cacc = lax.fori_loop(0, nch // 2, chunk_pair, (zero, zero))

    stage[...] = lacc
    pltpu.sync_copy(stage, loss_out.at[wid])
    stage[...] = cacc
    pltpu.sync_copy(stage, corr_out.at[wid])


def _sc_partial_sums(fc1, lb1, use_arc_f):
    N = fc1.shape[0]
    per_w = N // NW
    CH = 16000
    mesh = plsc.VectorSubcoreMesh(core_axis_name="c", subcore_axis_name="s")
    arc_vec = jnp.full((L,), use_arc_f, jnp.float32)

    kfn = pl.kernel(
        functools.partial(_sc_body, per_w=per_w, ch=CH),
        mesh=mesh,
        out_type=[
            jax.ShapeDtypeStruct((NW, L), jnp.float32),
            jax.ShapeDtypeStruct((NW, L), jnp.float32),
        ],
        scratch_types=[
            pltpu.VMEM((CH,), jnp.float32),
            pltpu.VMEM((CH,), jnp.float32),
            pltpu.VMEM((CH,), jnp.float32),
            pltpu.VMEM((CH,), jnp.float32),
            pltpu.VMEM((L,), jnp.float32),
            pltpu.VMEM((L,), jnp.float32),
            pltpu.SemaphoreType.DMA,
            pltpu.SemaphoreType.DMA,
        ],
    )
    return kfn(arc_vec, fc1, lb1)


def kernel(fc, label, epoch):
    B, C = fc.shape
    use_arc_f = (jnp.asarray(epoch, jnp.int32) >= ARC_START_EPOCH).astype(jnp.float32)
    loss_p, corr_p = _sc_partial_sums(fc.reshape(-1), label.reshape(-1), use_arc_f)
    inv_n = 1.0 / (B * C)
    focal = jnp.sum(loss_p) * inv_n
    acc = jnp.sum(corr_p) * inv_n
    return (focal, acc, focal)


# speedup vs baseline: 1.1385x; 1.1385x over previous
import functools

import jax
import jax.numpy as jnp
import numpy as np
from jax import lax
from jax.experimental import pallas as pl
from jax.experimental.pallas import tpu as pltpu
from jax.experimental.pallas import tpu_sc as plsc

S = 30.0
M = 0.5
ARC_START_EPOCH = 1
COS_M = float(np.cos(M))
SIN_M = float(np.sin(M))
BORDER = float(np.cos(np.pi - M))

_LOG1P_COEF = [float(c) for c in
               np.polynomial.chebyshev.Chebyshev.interpolate(
                   np.log1p, 7, domain=[0.0, 1.0])
               .convert(kind=np.polynomial.Polynomial).coef]

NW = 32
L = 16


def _sc_loss_tile(c, t, arc_b, scale_v):
    x = jnp.maximum(1.0 - c * c, 0.0)
    i = lax.bitcast_convert_type(x, jnp.int32)
    i = jnp.int32(0x5F3759DF) - lax.shift_right_arithmetic(i, 1)
    y = lax.bitcast_convert_type(i, jnp.float32)
    hx = 0.5 * x
    y = y * (1.5 - hx * y * y)
    y = y * (1.5 - hx * y * y)
    s = x * y

    phai = c * COS_M - s * SIN_M
    phai = jnp.where(c > BORDER, phai, -2.0 - phai)

    tmask = t != 0.0
    inner = jnp.where(arc_b, phai, c)
    v = scale_v * jnp.where(tmask, -inner, c)

    q = jnp.exp(jnp.minimum(v, -v))
    p = _LOG1P_COEF[7]
    for k in range(6, -1, -1):
        p = p * q + _LOG1P_COEF[k]
    sp = jnp.maximum(v, 0.0) + p
    loss = jnp.exp(2.0 * (v - sp)) * sp
    corr = jnp.where(v < 0.0, 1.0, 0.0)
    return loss, corr


def _sc_body(arc_hbm, scale_hbm, fc_hbm, lb_hbm, loss_out, corr_out,
             fcv0, lbv0, fcv1, lbv1, pvec, stage, sem0, sem1,
             *, rows_per_w, rch, ncols):
    cid = lax.axis_index("c")
    sid = lax.axis_index("s")
    wid = sid * 2 + cid
    row0 = wid * rows_per_w
    nch = rows_per_w // rch

    pltpu.sync_copy(arc_hbm, pvec)
    arc_b = pvec[...] != 0.0
    scale_v = jnp.where(arc_b, jnp.float32(S), jnp.float32(1.0))

    bufs = ((fcv0, lbv0, sem0), (fcv1, lbv1, sem1))

    def start(g, b):
        r = row0 + g * rch
        pltpu.async_copy(fc_hbm.at[pl.ds(r, rch), :], bufs[b][0], bufs[b][2])
        pltpu.async_copy(lb_hbm.at[pl.ds(r, rch), :], bufs[b][1], bufs[b][2])

    def wait(b):
        pltpu.make_async_copy(fc_hbm.at[pl.ds(0, rch), :], bufs[b][0],
                              bufs[b][2]).wait()
        pltpu.make_async_copy(lb_hbm.at[pl.ds(0, rch), :], bufs[b][1],
                              bufs[b][2]).wait()

    nfull = ncols // L
    tail0 = ncols - L
    taillo = nfull * L - tail0

    def compute(b, lacc, cacc):
        fcv, lbv = bufs[b][0], bufs[b][1]
        tail_mask = lax.iota(jnp.int32, L) >= taillo

        def row_step(r, carry):
            la, ca = carry

            def col_step(j, carry2):
                la2, ca2 = carry2
                c = fcv[r, pl.ds(j * L, L)]
                t = lbv[r, pl.ds(j * L, L)]
                lo, co = _sc_loss_tile(c, t, arc_b, scale_v)
                return la2 + lo, ca2 + co

            la, ca = lax.fori_loop(0, nfull, col_step, (la, ca), unroll=4)
            c = fcv[r, pl.ds(tail0, L)]
            t = lbv[r, pl.ds(tail0, L)]
            lo, co = _sc_loss_tile(c, t, arc_b, scale_v)
            la = la + jnp.where(tail_mask, lo, 0.0)
            ca = ca + jnp.where(tail_mask, co, 0.0)
            return la, ca

        return lax.fori_loop(0, rch, row_step, (lacc, cacc))

    zero = jnp.zeros((L,), jnp.float32)
    start(0, 0)

    def chunk_pair(i2, carry):
        lacc, cacc = carry
        g = i2 * 2
        wait(0)

        @pl.when(g + 1 < nch)
        def _():
            start(g + 1, 1)

        lacc, cacc = compute(0, lacc, cacc)
        wait(1)

        @pl.when(g + 2 < nch)
        def _():
            start(g + 2, 0)

        lacc, cacc = compute(1, lacc, cacc)
        return lacc, cacc

    lacc, cacc = lax.fori_loop(0, nch // 2, chunk_pair, (zero, zero))

    stage[...] = lacc
    pltpu.sync_copy(stage, loss_out.at[wid])
    stage[...] = cacc
    pltpu.sync_copy(stage, corr_out.at[wid])


def _sc_partial_sums(fc, label, use_arc_f):
    B, C = fc.shape
    rows_per_w = B // NW
    RCH = 16
    mesh = plsc.VectorSubcoreMesh(core_axis_name="c", subcore_axis_name="s")
    arc_vec = jnp.full((L,), use_arc_f, jnp.float32)

    kfn = pl.kernel(
        functools.partial(_sc_body, rows_per_w=rows_per_w, rch=RCH, ncols=C),
        mesh=mesh,
        out_type=[
            jax.ShapeDtypeStruct((NW, L), jnp.float32),
            jax.ShapeDtypeStruct((NW, L), jnp.float32),
        ],
        scratch_types=[
            pltpu.VMEM((RCH, C), jnp.float32),
            pltpu.VMEM((RCH, C), jnp.float32),
            pltpu.VMEM((RCH, C), jnp.float32),
            pltpu.VMEM((RCH, C), jnp.float32),
            pltpu.VMEM((L,), jnp.float32),
            pltpu.VMEM((L,), jnp.float32),
            pltpu.SemaphoreType.DMA,
            pltpu.SemaphoreType.DMA,
        ],
    )
    return kfn(arc_vec, arc_vec, fc, label)


def kernel(fc, label, epoch):
    B, C = fc.shape
    use_arc_f = (jnp.asarray(epoch, jnp.int32) >= ARC_START_EPOCH).astype(jnp.float32)
    loss_p, corr_p = _sc_partial_sums(fc, label, use_arc_f)
    inv_n = 1.0 / (B * C)
    focal = jnp.sum(loss_p) * inv_n
    acc = jnp.sum(corr_p) * inv_n
    return (focal, acc, focal)


# speedup vs baseline: 2.8704x; 2.5212x over previous
import functools

import jax
import jax.numpy as jnp
import numpy as np
from jax.experimental import pallas as pl
from jax.experimental.pallas import tpu as pltpu

S = 30.0
M = 0.5
ARC_START_EPOCH = 1
GAMMA = 2.0
COS_M = float(np.cos(M))
SIN_M = float(np.sin(M))
BORDER = float(np.cos(np.pi - M))


def _loss_body(use_arc_ref, scale_ref, fc_ref, label_ref, focal_ref, acc_ref,
               lacc_ref, cacc_ref, *, inv_n, rows, rsub, unroll):
    i = pl.program_id(0)
    nsteps = pl.num_programs(0)
    use_arc = use_arc_ref[0, 0] != 0
    scale = scale_ref[0, 0]

    @pl.when(i == 0)
    def _init():
        lacc_ref[...] = jnp.zeros_like(lacc_ref)
        cacc_ref[...] = jnp.zeros_like(cacc_ref)

    def tile(k):
        c = fc_ref[pl.ds(k * rsub, rsub), :]
        t = label_ref[pl.ds(k * rsub, rsub), :]

        sin_t = jnp.sqrt(jnp.maximum(1.0 - c * c, 0.0))
        phai = c * COS_M - sin_t * SIN_M
        phai = jnp.where(c > BORDER, phai, -2.0 - phai)

        tmask = t != 0.0
        inner = jnp.where(use_arc, phai, c)
        v = scale * jnp.where(tmask, -inner, c)

        log2e = 1.4426950408889634
        ln2 = 0.6931471805599453
        q = jnp.exp2(jnp.minimum(v, -v) * log2e)
        sp = jnp.maximum(v, 0.0) + jnp.log2(1.0 + q) * ln2
        loss = jnp.exp2((2.0 * log2e) * (v - sp)) * sp

        corr = jnp.where(v < 0.0, 1.0, 0.0)
        return loss, corr

    def group_step(g, carry):
        parts = [tile(g * unroll + j) for j in range(unroll)]
        ls = [p[0] for p in parts]
        cs = [p[1] for p in parts]
        while len(ls) > 1:
            ls = [a + b for a, b in zip(ls[::2], ls[1::2])]
            cs = [a + b for a, b in zip(cs[::2], cs[1::2])]
        lacc_ref[...] += ls[0]
        cacc_ref[...] += cs[0]
        return carry

    jax.lax.fori_loop(0, rows // rsub // unroll, group_step, 0)

    @pl.when(i == nsteps - 1)
    def _fin():
        focal_ref[0, 0] = jnp.sum(lacc_ref[...]) * inv_n
        acc_ref[0, 0] = jnp.sum(cacc_ref[...]) * inv_n


def kernel(fc, label, epoch):
    B, C = fc.shape
    BR = 512
    RSUB = 8
    UNROLL = 32
    nb = B // BR
    use_arc = (jnp.asarray(epoch, jnp.int32) >= ARC_START_EPOCH).astype(jnp.int32)
    scale = jnp.where(use_arc != 0, jnp.float32(S), jnp.float32(1.0))

    focal2d, acc2d = pl.pallas_call(
        functools.partial(_loss_body, inv_n=1.0 / (B * C),
                          rows=BR, rsub=RSUB, unroll=UNROLL),
        grid=(nb,),
        in_specs=[
            pl.BlockSpec(memory_space=pltpu.SMEM),
            pl.BlockSpec(memory_space=pltpu.SMEM),
            pl.BlockSpec((BR, C), lambda i: (i, 0)),
            pl.BlockSpec((BR, C), lambda i: (i, 0)),
        ],
        out_specs=[
            pl.BlockSpec(memory_space=pltpu.SMEM),
            pl.BlockSpec(memory_space=pltpu.SMEM),
        ],
        out_shape=[
            jax.ShapeDtypeStruct((1, 1), jnp.float32),
            jax.ShapeDtypeStruct((1, 1), jnp.float32),
        ],
        scratch_shapes=[
            pltpu.VMEM((RSUB, C), jnp.float32),
            pltpu.VMEM((RSUB, C), jnp.float32),
        ],
    )(use_arc.reshape(1, 1), scale.reshape(1, 1), fc, label)

    focal = focal2d[0, 0]
    acc = acc2d[0, 0]
    return (focal, acc, focal)
